# baseline (speedup 1.0000x reference)
import functools

import jax
import jax.numpy as jnp
from jax import lax
from jax.experimental import pallas as pl
from jax.experimental.pallas import tpu as pltpu
from jax.experimental.pallas import tpu_sc as plsc

VOCAB = 1000000
D = 32
BATCH = 16384
FIELDS = 26

NC = 2
NS = 16
NW = NC * NS

B_TOTAL = BATCH * FIELDS
B_PER_W = B_TOTAL // NW
CHUNK = 832
NCHUNKS = B_PER_W // CHUNK
assert NCHUNKS * CHUNK == B_PER_W
NBUF = 4


def _emb_body(x_hbm, lut_hbm, out_hbm, idx_v, bufs, gsems, wsems):
    cid = lax.axis_index("c")
    sid = lax.axis_index("s")
    wid = sid * NC + cid
    pltpu.sync_copy(x_hbm.at[wid], idx_v)

    def gather(c):
        return pltpu.async_copy(
            lut_hbm.at[idx_v.at[pl.ds(c * CHUNK, CHUNK)]],
            bufs[c % NBUF],
            gsems[c % NBUF],
        )

    def write(c):
        return pltpu.async_copy(bufs[c % NBUF], out_hbm.at[wid, c], wsems[c % NBUF])

    g = [None] * NCHUNKS
    w = [None] * NCHUNKS
    waited = [False] * NCHUNKS
    for c in range(min(NBUF, NCHUNKS)):
        g[c] = gather(c)
    for c in range(NCHUNKS):
        f = c + NBUF - 1
        if c > 0 and f < NCHUNKS:
            w[c - 1].wait()
            waited[c - 1] = True
            g[f] = gather(f)
        g[c].wait()
        w[c] = write(c)
    for c in range(NCHUNKS):
        if not waited[c]:
            w[c].wait()


_emb = functools.partial(
    pl.kernel,
    out_type=jax.ShapeDtypeStruct((NW, NCHUNKS, CHUNK, D), jnp.float32),
    mesh=plsc.VectorSubcoreMesh(core_axis_name="c", subcore_axis_name="s"),
    scratch_types=[
        pltpu.VMEM((B_PER_W,), jnp.int32),
        [pltpu.VMEM((CHUNK, D), jnp.float32) for _ in range(NBUF)],
        [pltpu.SemaphoreType.DMA for _ in range(NBUF)],
        [pltpu.SemaphoreType.DMA for _ in range(NBUF)],
    ],
    compiler_params=pltpu.CompilerParams(use_tc_tiling_on_sc=False),
)(_emb_body)


@jax.jit
def kernel(x, lut):
    xi = x.reshape(NW, B_PER_W).astype(jnp.int32)
    out = _emb(xi, lut)
    return out.reshape(BATCH, FIELDS, D)


# speedup vs baseline: 1.0147x; 1.0147x over previous
import functools

import jax
import jax.numpy as jnp
from jax import lax
from jax.experimental import pallas as pl
from jax.experimental.pallas import tpu as pltpu
from jax.experimental.pallas import tpu_sc as plsc

VOCAB = 1000000
D = 32
BATCH = 16384
FIELDS = 26

NC = 2
NS = 16
NW = NC * NS

B_TOTAL = BATCH * FIELDS
B_PER_W = B_TOTAL // NW
CHUNK = 832
NCHUNKS = B_PER_W // CHUNK
assert NCHUNKS * CHUNK == B_PER_W
NBUF = 4


def _emb_body(x_hbm, lut_hbm, out_hbm, idx_v, bufs, gsems, wsems):
    cid = lax.axis_index("c")
    sid = lax.axis_index("s")
    wid = sid * NC + cid
    pltpu.sync_copy(x_hbm.at[wid], idx_v)

    def gather(c):
        return pltpu.async_copy(
            lut_hbm.at[idx_v.at[pl.ds(c * CHUNK, CHUNK)]],
            bufs[c % NBUF],
            gsems[c % NBUF],
        )

    def write(c):
        return pltpu.async_copy(bufs[c % NBUF], out_hbm.at[wid, c], wsems[c % NBUF])

    g = [None] * NCHUNKS
    w = [None] * NCHUNKS
    waited = [False] * NCHUNKS
    for c in range(min(NBUF, NCHUNKS)):
        g[c] = gather(c)
    for c in range(NCHUNKS):
        f = c + NBUF - 1
        if c > 0 and f < NCHUNKS:
            w[c - 1].wait()
            waited[c - 1] = True
            g[f] = gather(f)
        g[c].wait()
        w[c] = write(c)
    for c in range(NCHUNKS):
        if not waited[c]:
            w[c].wait()


_emb = functools.partial(
    pl.kernel,
    out_type=jax.ShapeDtypeStruct((NW, NCHUNKS, CHUNK, D), jnp.float32),
    mesh=plsc.VectorSubcoreMesh(core_axis_name="c", subcore_axis_name="s"),
    scratch_types=[
        pltpu.VMEM((B_PER_W,), jnp.int32),
        [pltpu.VMEM((CHUNK, D), jnp.float32) for _ in range(NBUF)],
        [pltpu.SemaphoreType.DMA for _ in range(NBUF)],
        [pltpu.SemaphoreType.DMA for _ in range(NBUF)],
    ],
    compiler_params=pltpu.CompilerParams(use_tc_tiling_on_sc=False),
)(_emb_body)


@jax.jit
def kernel(x, lut):
    xi = (x.astype(jnp.int32) * 4).reshape(NW, B_PER_W)
    lut_p = jnp.pad(lut, ((0, 0), (0, 3 * D)))
    out = _emb(xi, lut_p.reshape(4 * VOCAB, D))
    return out.reshape(BATCH, FIELDS, D)


# speedup vs baseline: 1.1472x; 1.1306x over previous
import functools

import jax
import jax.numpy as jnp
from jax import lax
from jax.experimental import pallas as pl
from jax.experimental.pallas import tpu as pltpu
from jax.experimental.pallas import tpu_sc as plsc

VOCAB = 1000000
D = 32
BATCH = 16384
FIELDS = 26

NC = 2
NS = 16
NW = NC * NS

B_PER_W = BATCH // NW * FIELDS
BT_PER_W = BATCH // NW // 128
ATOMS = FIELDS * BT_PER_W
TPAD = 129


def _emb_body(x_hbm, lut_hbm, out_hbm, xbuf, ibuf, gb0, gb1, tb0, tb1,
              gs0, gs1, ts0, ts1):
    cid = lax.axis_index("c")
    sid = lax.axis_index("s")
    wid = sid * NC + cid
    pltpu.sync_copy(x_hbm.at[pl.ds(wid * B_PER_W, B_PER_W)], xbuf)

    iota = lax.iota(jnp.int32, 16)
    v26 = iota * FIELDS
    for a in range(ATOMS):
        f, j = a // BT_PER_W, a % BT_PER_W
        for g in range(8):
            src = v26 + ((128 * j + 16 * g) * FIELDS + f)
            ibuf[pl.ds(a * 128 + g * 16, 16)] = plsc.load_gather(xbuf, [src])

    def gather(a, gb, gs):
        pltpu.make_async_copy(
            lut_hbm.at[ibuf.at[pl.ds(a * 128, 128)]], gb, gs
        ).start()

    def atom(a, gb, tb, gs, ts, not_first, has_next):
        @pl.when(not_first)
        def _():
            pltpu.make_async_copy(tb.at[pl.ds(0, 8), pl.ds(0, 128)],
                                  out_hbm.at[0, 0, 0], ts).wait()
            pltpu.make_async_copy(tb.at[pl.ds(0, 8), pl.ds(0, 128)],
                                  out_hbm.at[0, 0, 0], ts).wait()
            pltpu.make_async_copy(tb.at[pl.ds(0, 8), pl.ds(0, 128)],
                                  out_hbm.at[0, 0, 0], ts).wait()
            pltpu.make_async_copy(tb.at[pl.ds(0, 8), pl.ds(0, 128)],
                                  out_hbm.at[0, 0, 0], ts).wait()

        pltpu.make_async_copy(
            lut_hbm.at[ibuf.at[pl.ds(a * 128, 128)]], gb, gs
        ).wait()

        for bl in range(128):
            blv = jnp.full((16,), bl, jnp.int32)
            r0 = plsc.load_gather(gb, [blv, iota])
            r1 = plsc.load_gather(gb, [blv, iota + 16])
            plsc.store_scatter(tb, [iota, blv], r0)
            plsc.store_scatter(tb, [iota + 16, blv], r1)

        f = a // BT_PER_W
        bt = wid * BT_PER_W + (a - f * BT_PER_W)
        for dt in range(4):
            pltpu.make_async_copy(
                tb.at[pl.ds(dt * 8, 8), pl.ds(0, 128)],
                out_hbm.at[f, dt, bt], ts,
            ).start()

        @pl.when(has_next)
        def _():
            gather(a + 2, gb, gs)

    gather(0, gb0, gs0)
    gather(1, gb1, gs1)

    def body(i, _):
        a0 = 2 * i
        atom(a0, gb0, tb0, gs0, ts0, a0 >= 2, a0 + 2 < ATOMS)
        atom(a0 + 1, gb1, tb1, gs1, ts1, a0 + 1 >= 2, a0 + 3 < ATOMS)
        return 0

    lax.fori_loop(0, ATOMS // 2, body, 0)

    for tb, ts in ((tb0, ts0), (tb1, ts1)):
        for _ in range(4):
            pltpu.make_async_copy(tb.at[pl.ds(0, 8), pl.ds(0, 128)],
                                  out_hbm.at[0, 0, 0], ts).wait()


_emb = functools.partial(
    pl.kernel,
    out_type=jax.ShapeDtypeStruct((FIELDS, 4, 128, 8, 128), jnp.float32),
    mesh=plsc.VectorSubcoreMesh(core_axis_name="c", subcore_axis_name="s"),
    scratch_types=[
        pltpu.VMEM((B_PER_W,), jnp.int32),
        pltpu.VMEM((B_PER_W,), jnp.int32),
        pltpu.VMEM((128, D), jnp.float32),
        pltpu.VMEM((128, D), jnp.float32),
        pltpu.VMEM((D, TPAD), jnp.float32),
        pltpu.VMEM((D, TPAD), jnp.float32),
        pltpu.SemaphoreType.DMA,
        pltpu.SemaphoreType.DMA,
        pltpu.SemaphoreType.DMA,
        pltpu.SemaphoreType.DMA,
    ],
    compiler_params=pltpu.CompilerParams(
        use_tc_tiling_on_sc=False, needs_layout_passes=False
    ),
)(_emb_body)


@jax.jit
def kernel(x, lut):
    xi = (x.astype(jnp.int32) * 4).reshape(BATCH * FIELDS)
    lut_p = jnp.pad(lut, ((0, 0), (0, 3 * D)))
    out5 = _emb(xi, lut_p.reshape(4 * VOCAB, D))
    return out5.transpose(2, 4, 0, 1, 3).reshape(BATCH, FIELDS, D)
